# Initial kernel scaffold; baseline (speedup 1.0000x reference)
import jax
import jax.numpy as jnp
from jax.experimental import pallas as pl


def kernel(x_user, x_movie, edge_label_index):
    raise NotImplementedError("write your pallas kernel here")


# speedup vs baseline: 5.2947x; 5.2947x over previous
import functools

import jax
import jax.numpy as jnp
from jax import lax
from jax.experimental import pallas as pl
from jax.experimental.pallas import tpu as pltpu
from jax.experimental.pallas import tpu_sc as plsc

_NUM_CORES = 2
_NUM_SUBCORES = 16
_NW = _NUM_CORES * _NUM_SUBCORES
_LANES = 16

_CHUNK = 80


def _dot_chunk(u_ref, m_ref, o_ref, p_ref, chunk, d_feat):
    n_groups = chunk // _LANES
    n_k = d_feat // _LANES
    lane = lax.iota(jnp.int32, _LANES)

    def group_body(g, _):
        for j in range(_LANES):
            e = g * _LANES + j
            p = u_ref[e, pl.ds(0, _LANES)] * m_ref[e, pl.ds(0, _LANES)]
            for k in range(1, n_k):
                p = p + (u_ref[e, pl.ds(k * _LANES, _LANES)]
                         * m_ref[e, pl.ds(k * _LANES, _LANES)])
            p_ref[pl.ds(j * _LANES, _LANES)] = p
        tcol = lane * _LANES
        acc = plsc.load_gather(p_ref, [tcol])
        for l in range(1, _LANES):
            acc = acc + plsc.load_gather(p_ref, [tcol + l])
        o_ref[pl.ds(g * _LANES, _LANES)] = acc
        return 0

    lax.fori_loop(0, n_groups, group_body, 0)


def _make_sc_kernel(n_edges, d_feat):
    per_w = n_edges // _NW
    n_chunks = per_w // _CHUNK
    mesh = plsc.VectorSubcoreMesh(
        core_axis_name="c", subcore_axis_name="s")

    @functools.partial(
        pl.kernel,
        out_type=jax.ShapeDtypeStruct((n_edges,), jnp.float32),
        mesh=mesh,
        compiler_params=pltpu.CompilerParams(needs_layout_passes=False),
        scratch_types=dict(
            i0_v=pltpu.VMEM((2, _CHUNK), jnp.int32),
            i1_v=pltpu.VMEM((2, _CHUNK), jnp.int32),
            u_v=pltpu.VMEM((2, _CHUNK, d_feat), jnp.float32),
            m_v=pltpu.VMEM((2, _CHUNK, d_feat), jnp.float32),
            o_v=pltpu.VMEM((_CHUNK,), jnp.float32),
            p_v=pltpu.VMEM((_LANES * _LANES,), jnp.float32),
            sems=pltpu.SemaphoreType.DMA((2,)),
        ),
    )
    def edge_dot(xu_hbm, xm_hbm, i0_hbm, i1_hbm, out_hbm,
                 i0_v, i1_v, u_v, m_v, o_v, p_v, sems):
        wid = lax.axis_index("s") * _NUM_CORES + lax.axis_index("c")
        base = wid * per_w

        def fire(c, slot):
            cb = base + c * _CHUNK
            pltpu.sync_copy(i0_hbm.at[pl.ds(cb, _CHUNK)], i0_v.at[slot])
            pltpu.sync_copy(i1_hbm.at[pl.ds(cb, _CHUNK)], i1_v.at[slot])
            pltpu.async_copy(xu_hbm.at[i0_v.at[slot]], u_v.at[slot],
                             sems.at[slot])
            pltpu.async_copy(xm_hbm.at[i1_v.at[slot]], m_v.at[slot],
                             sems.at[slot])

        fire(0, 0)

        def chunk_body(c, _):
            slot = lax.rem(c, 2)
            nxt = lax.rem(c + 1, 2)

            @pl.when(c + 1 < n_chunks)
            def _():
                fire(c + 1, nxt)

            pltpu.make_async_copy(
                xu_hbm.at[i0_v.at[slot]], u_v.at[slot], sems.at[slot]).wait()
            pltpu.make_async_copy(
                xm_hbm.at[i1_v.at[slot]], m_v.at[slot], sems.at[slot]).wait()

            _dot_chunk(u_v.at[slot], m_v.at[slot], o_v, p_v, _CHUNK, d_feat)

            pltpu.sync_copy(o_v, out_hbm.at[pl.ds(base + c * _CHUNK, _CHUNK)])
            return 0

        lax.fori_loop(0, n_chunks, chunk_body, 0)

    return edge_dot


def kernel(x_user, x_movie, edge_label_index):
    n_edges = edge_label_index.shape[1]
    d_feat = x_user.shape[1]
    idx0 = edge_label_index[0]
    idx1 = edge_label_index[1]
    sc_kernel = _make_sc_kernel(n_edges, d_feat)
    return sc_kernel(x_user, x_movie, idx0, idx1)


# speedup vs baseline: 8.1282x; 1.5352x over previous
import functools

import jax
import jax.numpy as jnp
from jax import lax
from jax.experimental import pallas as pl
from jax.experimental.pallas import tpu as pltpu
from jax.experimental.pallas import tpu_sc as plsc

_NUM_CORES = 2
_NUM_SUBCORES = 16
_NW = _NUM_CORES * _NUM_SUBCORES
_LANES = 16

_CHUNK = 80
_NBUF = 4


def _dot_chunk(u_ref, m_ref, o_ref, p_ref, obase, chunk, d_feat):
    n_groups = chunk // _LANES
    n_k = d_feat // _LANES
    lane = lax.iota(jnp.int32, _LANES)
    tcol = lane * _LANES

    def group_body(g, _):
        for j in range(_LANES):
            e = g * _LANES + j
            p = u_ref[e, pl.ds(0, _LANES)] * m_ref[e, pl.ds(0, _LANES)]
            for k in range(1, n_k):
                p = p + (u_ref[e, pl.ds(k * _LANES, _LANES)]
                         * m_ref[e, pl.ds(k * _LANES, _LANES)])
            p_ref[pl.ds(j * _LANES, _LANES)] = p
        acc = plsc.load_gather(p_ref, [tcol])
        for l in range(1, _LANES):
            acc = acc + plsc.load_gather(p_ref, [tcol + l])
        o_ref[pl.ds(obase + g * _LANES, _LANES)] = acc
        return 0

    lax.fori_loop(0, n_groups, group_body, 0)


def _make_sc_kernel(n_edges, d_feat):
    per_w = n_edges // _NW
    n_chunks = per_w // _CHUNK
    mesh = plsc.VectorSubcoreMesh(
        core_axis_name="c", subcore_axis_name="s")

    @functools.partial(
        pl.kernel,
        out_type=jax.ShapeDtypeStruct((n_edges,), jnp.float32),
        mesh=mesh,
        compiler_params=pltpu.CompilerParams(needs_layout_passes=False),
        scratch_types=dict(
            i0_v=pltpu.VMEM((per_w,), jnp.int32),
            i1_v=pltpu.VMEM((per_w,), jnp.int32),
            u_v=pltpu.VMEM((_NBUF, _CHUNK, d_feat), jnp.float32),
            m_v=pltpu.VMEM((_NBUF, _CHUNK, d_feat), jnp.float32),
            o_v=pltpu.VMEM((per_w,), jnp.float32),
            p_v=pltpu.VMEM((_LANES * _LANES,), jnp.float32),
            sems=pltpu.SemaphoreType.DMA((_NBUF,)),
        ),
    )
    def edge_dot(xu_hbm, xm_hbm, i0_hbm, i1_hbm, out_hbm,
                 i0_v, i1_v, u_v, m_v, o_v, p_v, sems):
        wid = lax.axis_index("s") * _NUM_CORES + lax.axis_index("c")
        base = wid * per_w

        pltpu.sync_copy(i0_hbm.at[pl.ds(base, per_w)], i0_v)
        pltpu.sync_copy(i1_hbm.at[pl.ds(base, per_w)], i1_v)

        def fire(c, slot):
            cb = c * _CHUNK
            pltpu.async_copy(xu_hbm.at[i0_v.at[pl.ds(cb, _CHUNK)]],
                             u_v.at[slot], sems.at[slot])
            pltpu.async_copy(xm_hbm.at[i1_v.at[pl.ds(cb, _CHUNK)]],
                             m_v.at[slot], sems.at[slot])

        for c in range(_NBUF - 1):
            fire(c, c)

        def chunk_body(c, _):
            slot = lax.rem(c, _NBUF)

            @pl.when(c + _NBUF - 1 < n_chunks)
            def _():
                fire(c + _NBUF - 1, lax.rem(c + _NBUF - 1, _NBUF))

            pltpu.make_async_copy(
                xu_hbm.at[i0_v.at[pl.ds(0, _CHUNK)]], u_v.at[slot],
                sems.at[slot]).wait()
            pltpu.make_async_copy(
                xm_hbm.at[i1_v.at[pl.ds(0, _CHUNK)]], m_v.at[slot],
                sems.at[slot]).wait()

            _dot_chunk(u_v.at[slot], m_v.at[slot], o_v, p_v,
                       c * _CHUNK, _CHUNK, d_feat)
            return 0

        lax.fori_loop(0, n_chunks, chunk_body, 0)
        pltpu.sync_copy(o_v, out_hbm.at[pl.ds(base, per_w)])

    return edge_dot


def kernel(x_user, x_movie, edge_label_index):
    n_edges = edge_label_index.shape[1]
    d_feat = x_user.shape[1]
    idx0 = edge_label_index[0]
    idx1 = edge_label_index[1]
    sc_kernel = _make_sc_kernel(n_edges, d_feat)
    return sc_kernel(x_user, x_movie, idx0, idx1)
